# Initial kernel scaffold; baseline (speedup 1.0000x reference)
import jax
import jax.numpy as jnp
from jax.experimental import pallas as pl


def kernel(idx, emb_w, ln_g, ln_b, head_w, head_q_w, head_k_w):
    raise NotImplementedError("write your pallas kernel here")


# speedup vs baseline: 1.5296x; 1.5296x over previous
import functools

import jax
import jax.numpy as jnp
from jax.experimental import pallas as pl
from jax.experimental.pallas import tpu as pltpu

LN_EPS = 1e-5

_F32 = jnp.float32
_BF16 = jnp.bfloat16

_G = 16
_TT = 256
_VT = 512


def _gather_ln_body(idx_ref, *refs, g: int):
    emb_refs = refs[:g]
    g_ref, b_ref = refs[g], refs[g + 1]
    x32_ref, xbf_ref = refs[g + 2], refs[g + 3]
    rows = jnp.concatenate([emb_refs[j][0] for j in range(g)], axis=0)
    mu = jnp.mean(rows, axis=1, keepdims=True)
    d = rows - mu
    var = jnp.mean(d * d, axis=1, keepdims=True)
    y = d * jax.lax.rsqrt(var + LN_EPS) * g_ref[...] + b_ref[...]
    x32_ref[...] = y
    xbf_ref[...] = y.astype(_BF16)


def _gather_ln(idx_flat, emb_w, ln_g2, ln_b2):
    n, e = idx_flat.shape[0], emb_w.shape[1]
    grid = (n // _G,)

    def _row_map(j):
        return lambda i, idx_ref: (idx_ref[i * _G + j], 0, 0)

    emb3 = emb_w.reshape(emb_w.shape[0], 1, e)
    in_specs = [pl.BlockSpec((1, 1, e), _row_map(j)) for j in range(_G)]
    in_specs += [
        pl.BlockSpec((1, e), lambda i, idx_ref: (0, 0)),
        pl.BlockSpec((1, e), lambda i, idx_ref: (0, 0)),
    ]
    out_specs = [
        pl.BlockSpec((_G, e), lambda i, idx_ref: (i, 0)),
        pl.BlockSpec((_G, e), lambda i, idx_ref: (i, 0)),
    ]
    return pl.pallas_call(
        functools.partial(_gather_ln_body, g=_G),
        grid_spec=pltpu.PrefetchScalarGridSpec(
            num_scalar_prefetch=1,
            grid=grid,
            in_specs=in_specs,
            out_specs=out_specs,
        ),
        out_shape=[
            jax.ShapeDtypeStruct((n, e), _F32),
            jax.ShapeDtypeStruct((n, e), _BF16),
        ],
        compiler_params=pltpu.CompilerParams(
            dimension_semantics=("parallel",),
        ),
        name="gather_ln",
    )(idx_flat, *([emb3] * _G), ln_g2, ln_b2)


_CDIMS_11 = (((1,), (1,)), ((), ()))
_CDIMS_10 = (((1,), (0,)), ((), ()))


def _qkc_body(x_ref, wq_ref, wk_ref, c_ref, k_s, *, t: int, qk: int):
    k_s[...] = jax.lax.dot_general(
        x_ref[0], wk_ref[...], _CDIMS_11, preferred_element_type=_F32
    ).astype(_BF16)
    inv_qk = _F32(1.0 / qk)
    for i in range(t // _TT):
        sl = slice(i * _TT, (i + 1) * _TT)
        ki = (i + 1) * _TT
        qi = jax.lax.dot_general(
            x_ref[0, sl], wq_ref[...], _CDIMS_11, preferred_element_type=_F32
        ).astype(_BF16)
        ci = jax.lax.dot_general(
            qi, k_s[:ki], _CDIMS_11, preferred_element_type=_F32
        ) * inv_qk
        row = i * _TT + jax.lax.broadcasted_iota(jnp.int32, (_TT, ki), 0)
        col = jax.lax.broadcasted_iota(jnp.int32, (_TT, ki), 1)
        ci = jnp.where(row >= col, ci, _F32(0.0))
        c_ref[0, sl, :ki] = ci.astype(_BF16)
        if ki < t:
            c_ref[0, sl, ki:] = jnp.zeros((_TT, t - ki), _BF16)


def _qkc(xbf, wq, wk):
    b, t, e = xbf.shape
    qk = wq.shape[0]
    return pl.pallas_call(
        functools.partial(_qkc_body, t=t, qk=qk),
        grid=(b,),
        in_specs=[
            pl.BlockSpec((1, t, e), lambda i: (i, 0, 0)),
            pl.BlockSpec((qk, e), lambda i: (0, 0)),
            pl.BlockSpec((qk, e), lambda i: (0, 0)),
        ],
        out_specs=pl.BlockSpec((1, t, t), lambda i: (i, 0, 0)),
        out_shape=jax.ShapeDtypeStruct((b, t, t), _BF16),
        scratch_shapes=[pltpu.VMEM((t, qk), _BF16)],
        compiler_params=pltpu.CompilerParams(
            dimension_semantics=("parallel",),
        ),
        name="qkc",
    )(xbf, wq, wk)


def _logits_body(idx_ref, x_ref, c_ref, hw_ref, o_ref, oh_s, *, b: int, t: int):
    v = pl.program_id(0)
    cols = v * _VT + jax.lax.broadcasted_iota(jnp.int32, (1, _VT), 1)
    hw = hw_ref[...].astype(_BF16)
    for bi in range(b):
        ids = idx_ref[bi]
        oh_s[...] = jnp.where(ids == cols, _F32(1.0), _F32(0.0)).astype(_BF16)
        for ti in range(t // _TT):
            sl = slice(ti * _TT, (ti + 1) * _TT)
            ki = (ti + 1) * _TT
            head_i = jax.lax.dot_general(
                x_ref[bi, sl], hw, _CDIMS_11, preferred_element_type=_F32
            )
            cpy_i = jax.lax.dot_general(
                c_ref[bi, sl, :ki], oh_s[:ki], _CDIMS_10,
                preferred_element_type=_F32,
            )
            o_ref[bi, sl] = head_i + cpy_i


def _logits(idx_col, xbf, c, head_w):
    b, t, e = xbf.shape
    v = head_w.shape[0]
    nv = (v + _VT - 1) // _VT
    return pl.pallas_call(
        functools.partial(_logits_body, b=b, t=t),
        grid=(nv,),
        in_specs=[
            pl.BlockSpec((b, t, 1), lambda i: (0, 0, 0)),
            pl.BlockSpec((b, t, e), lambda i: (0, 0, 0)),
            pl.BlockSpec((b, t, t), lambda i: (0, 0, 0)),
            pl.BlockSpec((_VT, e), lambda i: (i, 0)),
        ],
        out_specs=pl.BlockSpec((b, t, _VT), lambda i: (0, 0, i)),
        out_shape=jax.ShapeDtypeStruct((b, t, v), _F32),
        scratch_shapes=[pltpu.VMEM((t, _VT), _BF16)],
        compiler_params=pltpu.CompilerParams(
            dimension_semantics=("parallel",),
            vmem_limit_bytes=58 * 1024 * 1024,
        ),
        name="logits_copy",
    )(idx_col, xbf, c, head_w)


def kernel(idx, emb_w, ln_g, ln_b, head_w, head_q_w, head_k_w):
    b, t = idx.shape
    e = emb_w.shape[1]
    idx = idx.astype(jnp.int32)
    x32f, xbff = _gather_ln(
        idx.reshape(-1), emb_w, ln_g.reshape(1, e), ln_b.reshape(1, e)
    )
    x = x32f.reshape(b, t, e)
    xbf = xbff.reshape(b, t, e)
    c = _qkc(xbf, head_q_w.astype(_BF16), head_k_w.astype(_BF16))
    logits = _logits(idx[:, :, None], xbf, c, head_w)
    return logits, x


# speedup vs baseline: 3.2984x; 2.1564x over previous
import functools

import jax
import jax.numpy as jnp
from jax.experimental import pallas as pl
from jax.experimental.pallas import tpu as pltpu

LN_EPS = 1e-5

_F32 = jnp.float32
_BF16 = jnp.bfloat16

_G = 512
_TT = 256
_VT = 512


def _gather_ln_body(idx_ref, emb_ref, g_ref, b_ref, x32_ref, xbf_ref,
                    xg, sem, *, g: int):
    t0 = pl.program_id(0) * g
    for r in range(g):
        pltpu.make_async_copy(
            emb_ref.at[idx_ref[t0 + r]], xg.at[r], sem
        ).start()
    for r in range(g):
        pltpu.make_async_copy(
            emb_ref.at[idx_ref[t0 + r]], xg.at[r], sem
        ).wait()
    rows = xg[...]
    mu = jnp.mean(rows, axis=1, keepdims=True)
    d = rows - mu
    var = jnp.mean(d * d, axis=1, keepdims=True)
    y = d * jax.lax.rsqrt(var + LN_EPS) * g_ref[...] + b_ref[...]
    x32_ref[...] = y
    xbf_ref[...] = y.astype(_BF16)


def _gather_ln(idx_flat, emb_w, ln_g2, ln_b2):
    n, e = idx_flat.shape[0], emb_w.shape[1]
    grid = (n // _G,)
    in_specs = [
        pl.BlockSpec(memory_space=pl.ANY),
        pl.BlockSpec((1, e), lambda i, idx_ref: (0, 0)),
        pl.BlockSpec((1, e), lambda i, idx_ref: (0, 0)),
    ]
    out_specs = [
        pl.BlockSpec((_G, e), lambda i, idx_ref: (i, 0)),
        pl.BlockSpec((_G, e), lambda i, idx_ref: (i, 0)),
    ]
    return pl.pallas_call(
        functools.partial(_gather_ln_body, g=_G),
        grid_spec=pltpu.PrefetchScalarGridSpec(
            num_scalar_prefetch=1,
            grid=grid,
            in_specs=in_specs,
            out_specs=out_specs,
            scratch_shapes=[
                pltpu.VMEM((_G, e), _F32),
                pltpu.SemaphoreType.DMA,
            ],
        ),
        out_shape=[
            jax.ShapeDtypeStruct((n, e), _F32),
            jax.ShapeDtypeStruct((n, e), _BF16),
        ],
        compiler_params=pltpu.CompilerParams(
            dimension_semantics=("arbitrary",),
        ),
        name="gather_ln",
    )(idx_flat, emb_w, ln_g2, ln_b2)


_CDIMS_11 = (((1,), (1,)), ((), ()))
_CDIMS_10 = (((1,), (0,)), ((), ()))


def _qkc_body(x_ref, wq_ref, wk_ref, c_ref, k_s, *, t: int, qk: int):
    k_s[...] = jax.lax.dot_general(
        x_ref[0], wk_ref[...], _CDIMS_11, preferred_element_type=_F32
    ).astype(_BF16)
    inv_qk = _F32(1.0 / qk)
    for i in range(t // _TT):
        sl = slice(i * _TT, (i + 1) * _TT)
        ki = (i + 1) * _TT
        qi = jax.lax.dot_general(
            x_ref[0, sl], wq_ref[...], _CDIMS_11, preferred_element_type=_F32
        ).astype(_BF16)
        ci = jax.lax.dot_general(
            qi, k_s[:ki], _CDIMS_11, preferred_element_type=_F32
        ) * inv_qk
        row = i * _TT + jax.lax.broadcasted_iota(jnp.int32, (_TT, ki), 0)
        col = jax.lax.broadcasted_iota(jnp.int32, (_TT, ki), 1)
        ci = jnp.where(row >= col, ci, _F32(0.0))
        c_ref[0, sl, :ki] = ci.astype(_BF16)
        if ki < t:
            c_ref[0, sl, ki:] = jnp.zeros((_TT, t - ki), _BF16)


def _qkc(xbf, wq, wk):
    b, t, e = xbf.shape
    qk = wq.shape[0]
    return pl.pallas_call(
        functools.partial(_qkc_body, t=t, qk=qk),
        grid=(b,),
        in_specs=[
            pl.BlockSpec((1, t, e), lambda i: (i, 0, 0)),
            pl.BlockSpec((qk, e), lambda i: (0, 0)),
            pl.BlockSpec((qk, e), lambda i: (0, 0)),
        ],
        out_specs=pl.BlockSpec((1, t, t), lambda i: (i, 0, 0)),
        out_shape=jax.ShapeDtypeStruct((b, t, t), _BF16),
        scratch_shapes=[pltpu.VMEM((t, qk), _BF16)],
        compiler_params=pltpu.CompilerParams(
            dimension_semantics=("parallel",),
        ),
        name="qkc",
    )(xbf, wq, wk)


def _logits_body(idx_ref, x_ref, c_ref, hw_ref, o_ref, oh_s, *, b: int, t: int):
    v = pl.program_id(0)
    col_ids = v * _VT + jax.lax.broadcasted_iota(jnp.int32, (_VT, t), 0)
    hw = hw_ref[...].astype(_BF16)
    for bi in range(b):
        ids = idx_ref[bi]
        oh_s[...] = jnp.where(ids == col_ids, _F32(1.0), _F32(0.0)).astype(_BF16)
        for ti in range(t // _TT):
            sl = slice(ti * _TT, (ti + 1) * _TT)
            ki = (ti + 1) * _TT
            head_t = jax.lax.dot_general(
                hw, x_ref[bi, sl], _CDIMS_11, preferred_element_type=_F32
            )
            cpy_t = jax.lax.dot_general(
                oh_s[:, :ki], c_ref[bi, sl, :ki], _CDIMS_11,
                preferred_element_type=_F32,
            )
            o_ref[:, bi, sl] = head_t + cpy_t


def _logits(idx_row, xbf, c, head_w):
    b, t, e = xbf.shape
    v = head_w.shape[0]
    nv = (v + _VT - 1) // _VT
    out = pl.pallas_call(
        functools.partial(_logits_body, b=b, t=t),
        grid=(nv,),
        in_specs=[
            pl.BlockSpec((b, 1, t), lambda i: (0, 0, 0)),
            pl.BlockSpec((b, t, e), lambda i: (0, 0, 0)),
            pl.BlockSpec((b, t, t), lambda i: (0, 0, 0)),
            pl.BlockSpec((_VT, e), lambda i: (i, 0)),
        ],
        out_specs=pl.BlockSpec((_VT, b, t), lambda i: (i, 0, 0)),
        out_shape=jax.ShapeDtypeStruct((v, b, t), _F32),
        scratch_shapes=[pltpu.VMEM((_VT, t), _BF16)],
        compiler_params=pltpu.CompilerParams(
            dimension_semantics=("parallel",),
            vmem_limit_bytes=58 * 1024 * 1024,
        ),
        name="logits_copy",
    )(idx_row, xbf, c, head_w)
    return jnp.transpose(out, (1, 2, 0))


def kernel(idx, emb_w, ln_g, ln_b, head_w, head_q_w, head_k_w):
    b, t = idx.shape
    e = emb_w.shape[1]
    idx = idx.astype(jnp.int32)
    x32f, xbff = _gather_ln(
        idx.reshape(-1), emb_w, ln_g.reshape(1, e), ln_b.reshape(1, e)
    )
    x = x32f.reshape(b, t, e)
    xbf = xbff.reshape(b, t, e)
    c = _qkc(xbf, head_q_w.astype(_BF16), head_k_w.astype(_BF16))
    logits = _logits(idx.reshape(b, 1, t), xbf, c, head_w)
    return logits, x
